# baseline (speedup 1.0000x reference)
import jax
import jax.numpy as jnp
from jax.experimental import pallas as pl
from jax.experimental.pallas import tpu as pltpu
from jax.experimental.pallas import tpu_sc as plsc

_BW = 128
_NW = 32
_SLOTS = 5
_VBLK = 16384


def _widen(tableT, eye):
    D, V = tableT.shape
    nblk = pl.cdiv(V, _VBLK)

    def body(t_ref, e_ref, o_ref):
        o_ref[...] = jax.lax.dot_general(
            t_ref[...],
            e_ref[...],
            (((0,), (0,)), ((), ())),
            preferred_element_type=jnp.float32,
            precision=jax.lax.Precision.DEFAULT,
        )

    return pl.pallas_call(
        body,
        grid=(nblk,),
        compiler_params=pltpu.CompilerParams(
            fuse_transposed_lhs_in_matmul=True
        ),
        in_specs=[
            pl.BlockSpec((D, _VBLK), lambda i: (0, i)),
            pl.BlockSpec((D, 128), lambda i: (0, 0)),
        ],
        out_specs=pl.BlockSpec((_VBLK, 128), lambda i: (i, 0)),
        out_shape=jax.ShapeDtypeStruct((V, 128), tableT.dtype),
    )(tableT, eye)


def kernel(table, words_pretrained):
    V, D = table.shape
    B, L = words_pretrained.shape
    assert B == _NW * _BW and D == 64
    idx = words_pretrained.T.reshape(L, _NW, _BW).astype(jnp.int32) * 2
    eye = jnp.eye(D, 128, dtype=table.dtype)

    mesh = plsc.VectorSubcoreMesh(
        core_axis_name="core", subcore_axis_name="subcore"
    )

    @jax.jit
    def run(table, eye, idx):
        t128 = _widen(table.T, eye).reshape(2 * V, D)

        @pl.kernel(
            out_type=jax.ShapeDtypeStruct(
                (L, D // 8, _NW, 8, _BW), table.dtype
            ),
            mesh=mesh,
            compiler_params=pltpu.CompilerParams(
                use_tc_tiling_on_sc=False, needs_layout_passes=False
            ),
            scratch_types=[
                pltpu.VMEM((L, _BW), jnp.int32),
                pltpu.VMEM((_SLOTS * _BW, D), jnp.float32),
                pltpu.VMEM((_SLOTS * 8, 8, _BW + 1), jnp.float32),
                pltpu.SemaphoreType.DMA,
                pltpu.SemaphoreType.DMA((_SLOTS,)),
                pltpu.SemaphoreType.DMA((_SLOTS,)),
            ],
        )
        def k(x_hbm, i_hbm, o_hbm, idx_v, in_v, out_v, isem, gsem, ssem):
            wid = (
                jax.lax.axis_index("core") * 16
                + jax.lax.axis_index("subcore")
            )
            pltpu.async_copy(i_hbm.at[:, wid], idx_v, isem).wait()
            lane = jax.lax.iota(jnp.int32, 16)

            def gather(l, b):
                pltpu.async_copy(
                    x_hbm.at[idx_v.at[l]],
                    in_v.at[pl.ds(b * _BW, _BW)],
                    gsem.at[b],
                )

            def gather_wait(l, b):
                pltpu.make_async_copy(
                    x_hbm.at[idx_v.at[l]],
                    in_v.at[pl.ds(b * _BW, _BW)],
                    gsem.at[b],
                ).wait()

            def store(l, b):
                pltpu.async_copy(
                    out_v.at[pl.ds(b * 8, 8), :, pl.ds(0, _BW)],
                    o_hbm.at[l, :, wid],
                    ssem.at[b],
                )

            def store_wait(b):
                pltpu.make_async_copy(
                    out_v.at[pl.ds(b * 8, 8), :, pl.ds(0, _BW)],
                    o_hbm.at[0, :, wid],
                    ssem.at[b],
                ).wait()

            def transpose(b):
                i0 = [
                    (lane + j * 16) // 8 + jnp.int32(b * 8)
                    for j in range(D // 16)
                ]
                i1 = [(lane + j * 16) % 8 for j in range(D // 16)]

                @pl.loop(0, _BW, step=16)
                def _(r0):
                    for ri in range(16):
                        r = r0 + ri
                        rcol = jnp.full((16,), r, jnp.int32)
                        for j in range(D // 16):
                            v = in_v[b * _BW + r, pl.ds(j * 16, 16)]
                            plsc.store_scatter(
                                out_v, [i0[j], i1[j], rcol], v
                            )

            for b in range(_SLOTS):
                gather(b, b)

            @pl.loop(0, L, step=_SLOTS)
            def _(c):
                for b in range(_SLOTS):
                    gather_wait(c + b, b)

                    @pl.when(c > 0)
                    def _():
                        store_wait(b)

                    transpose(b)

                    @pl.when(c + _SLOTS + b < L)
                    def _():
                        gather(c + _SLOTS + b, b)

                    store(c + b, b)

            for b in range(_SLOTS):
                store_wait(b)

        return k(t128, idx)

    out5 = run(table, eye, idx)
    return out5.transpose(2, 4, 0, 1, 3).reshape(B, L, D)


# speedup vs baseline: 1.6301x; 1.6301x over previous
import jax
import jax.numpy as jnp
from jax.experimental import pallas as pl
from jax.experimental.pallas import tpu as pltpu
from jax.experimental.pallas import tpu_sc as plsc

_BW = 128
_NW = 32
_SLOTS = 5
_VBLK = 16384


def _widen(tableT, eye):
    D, V = tableT.shape
    nblk = pl.cdiv(V, _VBLK)

    def body(t_ref, e_ref, o_ref):
        o_ref[...] = jax.lax.dot_general(
            t_ref[...],
            e_ref[...],
            (((0,), (0,)), ((), ())),
            preferred_element_type=jnp.float32,
            precision=jax.lax.Precision.DEFAULT,
        )

    return pl.pallas_call(
        body,
        grid=(nblk,),
        compiler_params=pltpu.CompilerParams(
            fuse_transposed_lhs_in_matmul=True
        ),
        in_specs=[
            pl.BlockSpec((D, _VBLK), lambda i: (0, i)),
            pl.BlockSpec((D, 128), lambda i: (0, 0)),
        ],
        out_specs=pl.BlockSpec((_VBLK, 128), lambda i: (i, 0)),
        out_shape=jax.ShapeDtypeStruct((V, 128), tableT.dtype),
    )(tableT, eye)


def kernel(table, words_pretrained):
    V, D = table.shape
    B, L = words_pretrained.shape
    assert B == _NW * _BW and D == 64
    idx = words_pretrained.T.reshape(L, _NW, _BW).astype(jnp.int32) * 2
    eye = jnp.eye(D, 128, dtype=table.dtype)

    mesh = plsc.VectorSubcoreMesh(
        core_axis_name="core", subcore_axis_name="subcore"
    )

    @jax.jit
    def run(table, eye, idx):
        t128 = _widen(table.T, eye).reshape(2 * V, D)

        @pl.kernel(
            out_type=jax.ShapeDtypeStruct(
                (L, D // 8, _NW, 8, _BW), table.dtype
            ),
            mesh=mesh,
            compiler_params=pltpu.CompilerParams(
                use_tc_tiling_on_sc=False, needs_layout_passes=False
            ),
            scratch_types=[
                pltpu.VMEM((L, _BW), jnp.int32),
                pltpu.VMEM((_SLOTS * _BW, D), jnp.float32),
                pltpu.VMEM((_SLOTS * 8, 8, _BW + 1), jnp.float32),
                pltpu.SemaphoreType.DMA,
                pltpu.SemaphoreType.DMA((_SLOTS,)),
                pltpu.SemaphoreType.DMA((_SLOTS,)),
            ],
        )
        def k(x_hbm, i_hbm, o_hbm, idx_v, in_v, out_v, isem, gsem, ssem):
            wid = (
                jax.lax.axis_index("core") * 16
                + jax.lax.axis_index("subcore")
            )
            pltpu.async_copy(i_hbm.at[:, wid], idx_v, isem).wait()
            lane = jax.lax.iota(jnp.int32, 16)

            def gather(l, b):
                pltpu.async_copy(
                    x_hbm.at[idx_v.at[l]],
                    in_v.at[pl.ds(b * _BW, _BW)],
                    gsem.at[b],
                )

            def gather_wait(l, b):
                pltpu.make_async_copy(
                    x_hbm.at[idx_v.at[l]],
                    in_v.at[pl.ds(b * _BW, _BW)],
                    gsem.at[b],
                ).wait()

            def store(l, b):
                pltpu.async_copy(
                    out_v.at[pl.ds(b * 8, 8), :, pl.ds(0, _BW)],
                    o_hbm.at[l, :, wid],
                    ssem.at[b],
                )

            def store_wait(b):
                pltpu.make_async_copy(
                    out_v.at[pl.ds(b * 8, 8), :, pl.ds(0, _BW)],
                    o_hbm.at[0, :, wid],
                    ssem.at[b],
                ).wait()

            def transpose(b):
                i0 = [
                    (lane + j * 16) // 8 + jnp.int32(b * 8)
                    for j in range(D // 16)
                ]
                i1 = [(lane + j * 16) % 8 for j in range(D // 16)]

                @plsc.parallel_loop(0, _BW, unroll=8)
                def _(r):
                    rcol = jnp.full((16,), r, jnp.int32)
                    for j in range(D // 16):
                        v = in_v[b * _BW + r, pl.ds(j * 16, 16)]
                        plsc.store_scatter(
                            out_v, [i0[j], i1[j], rcol], v
                        )

            for b in range(_SLOTS):
                gather(b, b)

            @pl.loop(0, L, step=_SLOTS)
            def _(c):
                for b in range(_SLOTS):
                    gather_wait(c + b, b)

                    @pl.when(c > 0)
                    def _():
                        store_wait(b)

                    transpose(b)

                    @pl.when(c + _SLOTS + b < L)
                    def _():
                        gather(c + _SLOTS + b, b)

                    store(c + b, b)

            for b in range(_SLOTS):
                store_wait(b)

        return k(t128, idx)

    out5 = run(table, eye, idx)
    return out5.transpose(2, 4, 0, 1, 3).reshape(B, L, D)


# speedup vs baseline: 1.6338x; 1.0023x over previous
import jax
import jax.numpy as jnp
from jax.experimental import pallas as pl
from jax.experimental.pallas import tpu as pltpu
from jax.experimental.pallas import tpu_sc as plsc

_BW = 128
_NW = 32
_SLOTS = 5
_VBLK = 16384


def _widen(tableT, eye):
    D, V = tableT.shape
    nblk = pl.cdiv(V, _VBLK)

    def body(t_ref, e_ref, o_ref):
        o_ref[:, 0:D] = jax.lax.dot_general(
            t_ref[...],
            e_ref[...],
            (((0,), (0,)), ((), ())),
            preferred_element_type=jnp.float32,
            precision=jax.lax.Precision.DEFAULT,
        )

    return pl.pallas_call(
        body,
        grid=(nblk,),
        compiler_params=pltpu.CompilerParams(
            fuse_transposed_lhs_in_matmul=True
        ),
        in_specs=[
            pl.BlockSpec((D, _VBLK), lambda i: (0, i)),
            pl.BlockSpec((D, D), lambda i: (0, 0)),
        ],
        out_specs=pl.BlockSpec((_VBLK, 128), lambda i: (i, 0)),
        out_shape=jax.ShapeDtypeStruct((V, 128), tableT.dtype),
    )(tableT, eye)


def kernel(table, words_pretrained):
    V, D = table.shape
    B, L = words_pretrained.shape
    assert B == _NW * _BW and D == 64
    idx = words_pretrained.T.reshape(L, _NW, _BW).astype(jnp.int32) * 2
    eye = jnp.eye(D, dtype=table.dtype)

    mesh = plsc.VectorSubcoreMesh(
        core_axis_name="core", subcore_axis_name="subcore"
    )

    @jax.jit
    def run(table, eye, idx):
        t128 = _widen(table.T, eye).reshape(2 * V, D)

        @pl.kernel(
            out_type=jax.ShapeDtypeStruct(
                (L, D // 8, _NW, 8, _BW), table.dtype
            ),
            mesh=mesh,
            compiler_params=pltpu.CompilerParams(
                use_tc_tiling_on_sc=False, needs_layout_passes=False
            ),
            scratch_types=[
                pltpu.VMEM((L, _BW), jnp.int32),
                pltpu.VMEM((_SLOTS * _BW, D), jnp.float32),
                pltpu.VMEM((_SLOTS * 8, 8, _BW + 1), jnp.float32),
                pltpu.SemaphoreType.DMA,
                pltpu.SemaphoreType.DMA((_SLOTS,)),
                pltpu.SemaphoreType.DMA((_SLOTS,)),
            ],
        )
        def k(x_hbm, i_hbm, o_hbm, idx_v, in_v, out_v, isem, gsem, ssem):
            wid = (
                jax.lax.axis_index("core") * 16
                + jax.lax.axis_index("subcore")
            )
            pltpu.async_copy(i_hbm.at[:, wid], idx_v, isem).wait()
            lane = jax.lax.iota(jnp.int32, 16)

            def gather(l, b):
                pltpu.async_copy(
                    x_hbm.at[idx_v.at[l]],
                    in_v.at[pl.ds(b * _BW, _BW)],
                    gsem.at[b],
                )

            def gather_wait(l, b):
                pltpu.make_async_copy(
                    x_hbm.at[idx_v.at[l]],
                    in_v.at[pl.ds(b * _BW, _BW)],
                    gsem.at[b],
                ).wait()

            def store(l, b):
                pltpu.async_copy(
                    out_v.at[pl.ds(b * 8, 8), :, pl.ds(0, _BW)],
                    o_hbm.at[l, :, wid],
                    ssem.at[b],
                )

            def store_wait(b):
                pltpu.make_async_copy(
                    out_v.at[pl.ds(b * 8, 8), :, pl.ds(0, _BW)],
                    o_hbm.at[0, :, wid],
                    ssem.at[b],
                ).wait()

            def transpose(b):
                i0 = [
                    (lane + j * 16) // 8 + jnp.int32(b * 8)
                    for j in range(D // 16)
                ]
                i1 = [(lane + j * 16) % 8 for j in range(D // 16)]

                @plsc.parallel_loop(0, _BW, unroll=8)
                def _(r):
                    rcol = jnp.full((16,), r, jnp.int32)
                    for j in range(D // 16):
                        v = in_v[b * _BW + r, pl.ds(j * 16, 16)]
                        plsc.store_scatter(
                            out_v, [i0[j], i1[j], rcol], v
                        )

            for b in range(_SLOTS):
                gather(b, b)

            @pl.loop(0, L, step=_SLOTS)
            def _(c):
                for b in range(_SLOTS):
                    gather_wait(c + b, b)

                    @pl.when(c > 0)
                    def _():
                        store_wait(b)

                    transpose(b)

                    @pl.when(c + _SLOTS + b < L)
                    def _():
                        gather(c + _SLOTS + b, b)

                    store(c + b, b)

            for b in range(_SLOTS):
                store_wait(b)

        return k(t128, idx)

    out5 = run(table, eye, idx)
    return out5.transpose(2, 4, 0, 1, 3).reshape(B, L, D)
